# baseline (speedup 1.0000x reference)
import functools
import jax
import jax.numpy as jnp
from jax import lax
from jax.experimental import pallas as pl
from jax.experimental.pallas import tpu as pltpu
from jax.experimental.pallas import tpu_sc as plsc

N = 10000
NP = 10112
D = 128
CH = 112
NB = 6
NBLK0 = 21
NBLK1 = 9
NBLKD = 15
NBLKS = 16 * (NBLK0 + NBLK1)
NBLKS_PAD = NBLKS + 16
E = 320000
EP = NBLKS * NB * CH
RPT = NP // 16

_mesh = plsc.VectorSubcoreMesh(core_axis_name="c", subcore_axis_name="s")


@functools.partial(
    pl.kernel,
    out_type=jax.ShapeDtypeStruct((2 * NP, D), jnp.float32),
    mesh=_mesh,
    scratch_types=[
        pltpu.VMEM((2, 2, NB, CH), jnp.int32),
        pltpu.VMEM((CH, D), jnp.float32),
        pltpu.VMEM((CH, D), jnp.float32),
        pltpu.VMEM_SHARED((NP, D), jnp.float32),
        pltpu.SemaphoreType.DMA,
        pltpu.SemaphoreType.DMA,
        pltpu.SemaphoreType.DMA,
    ],
)
def _spmv_kernel(y_hbm, rc_hbm, out_hbm, rcblk, gbufa, gbufb,
                 z_s, sema, semb, semr):
    c = lax.axis_index("c")
    s = lax.axis_index("s")
    nblk = jnp.where(c == 0, NBLK0, NBLK1)
    blkbase = jnp.where(c == 0, s * NBLK0, 16 * NBLK0 + s * NBLK1)

    zero16 = jnp.zeros((16,), jnp.float32)

    @pl.loop(0, CH)
    def _(i):
        for q in range(8):
            gbufa[i, pl.ds(16 * q, 16)] = zero16

    @pl.loop(0, 5)
    def _(k):
        pltpu.sync_copy(gbufa.at[pl.ds(0, 112)],
                        z_s.at[pl.ds(s * RPT + k * 112, 112)])

    pltpu.sync_copy(gbufa.at[pl.ds(0, 72)], z_s.at[pl.ds(s * RPT + 560, 72)])

    plsc.subcore_barrier()

    gbufs = (gbufa, gbufb)
    sems = (sema, semb)

    def wait_gather(buf, sem):
        pltpu.make_async_copy(y_hbm.at[rcblk.at[0, 0, 0]], buf, sem).wait()

    def wait_rcblk():
        pltpu.make_async_copy(rc_hbm.at[blkbase], rcblk.at[0], semr).wait()

    pltpu.sync_copy(rc_hbm.at[blkbase], rcblk.at[0])
    pltpu.async_copy(rc_hbm.at[blkbase + 1], rcblk.at[1], semr)
    pltpu.async_copy(y_hbm.at[rcblk.at[0, 0, 0]], gbufa, sema)

    def block(b, cur, nxt):
        for t in range(NB):
            if t < NB - 1:
                pltpu.async_copy(y_hbm.at[rcblk.at[cur, 0, t + 1]],
                                 gbufs[(t + 1) % 2], sems[(t + 1) % 2])
            else:
                @pl.when(b < nblk - 1)
                def _():
                    wait_rcblk()
                    pltpu.async_copy(y_hbm.at[rcblk.at[nxt, 0, 0]],
                                     gbufs[(t + 1) % 2], sems[(t + 1) % 2])
            wait_gather(gbufs[t % 2], sems[t % 2])
            pltpu.sync_copy(gbufs[t % 2], z_s.at[rcblk.at[cur, 1, t]],
                            add=True)

        @pl.when(b + 2 < nblk)
        def _():
            pltpu.async_copy(rc_hbm.at[blkbase + b + 2], rcblk.at[cur], semr)

    @pl.loop(0, (nblk - 1) // 2)
    def _(q):
        block(2 * q, 0, 1)
        block(2 * q + 1, 1, 0)

    block(nblk - 1, 0, 1)

    plsc.subcore_barrier()
    pltpu.sync_copy(
        z_s.at[pl.ds(s * RPT, RPT)],
        out_hbm.at[pl.ds(c * NP + s * RPT, RPT)],
    )


@functools.partial(
    pl.kernel,
    out_type=jax.ShapeDtypeStruct((2 * NP, D), jnp.float32),
    mesh=_mesh,
    scratch_types=[
        pltpu.VMEM((2, 2, NB, CH), jnp.int32),
        pltpu.VMEM((CH, D), jnp.float32),
        pltpu.VMEM_SHARED((NP, D), jnp.float32),
        pltpu.SemaphoreType.DMA,
    ],
)
def _deg_kernel(rc_hbm, out_hbm, rcblk, gones, z_s, semr):
    c = lax.axis_index("c")
    s = lax.axis_index("s")
    nblk = NBLKD
    blkbase = (c * 16 + s) * NBLKD

    zero16 = jnp.zeros((16,), jnp.float32)
    ones16 = jnp.ones((16,), jnp.float32)

    @pl.loop(0, CH)
    def _(i):
        for q in range(8):
            gones[i, pl.ds(16 * q, 16)] = zero16

    @pl.loop(0, 5)
    def _(k):
        pltpu.sync_copy(gones.at[pl.ds(0, 112)],
                        z_s.at[pl.ds(s * RPT + k * 112, 112)])

    pltpu.sync_copy(gones.at[pl.ds(0, 72)], z_s.at[pl.ds(s * RPT + 560, 72)])

    @pl.loop(0, CH)
    def _(i):
        gones[i, pl.ds(0, 16)] = ones16

    plsc.subcore_barrier()

    def wait_rcblk():
        pltpu.make_async_copy(rc_hbm.at[blkbase], rcblk.at[0], semr).wait()

    pltpu.sync_copy(rc_hbm.at[blkbase], rcblk.at[0])
    pltpu.async_copy(rc_hbm.at[blkbase + 1], rcblk.at[1], semr)

    def block(b, cur, nxt):
        for t in range(NB):
            pltpu.sync_copy(gones, z_s.at[rcblk.at[cur, 1, t]], add=True)

        @pl.when(b < nblk - 2)
        def _():
            wait_rcblk()
            pltpu.async_copy(rc_hbm.at[blkbase + b + 2], rcblk.at[cur], semr)

        @pl.when(b == nblk - 2)
        def _():
            wait_rcblk()

    @pl.loop(0, NBLKD // 2)
    def _(q):
        block(2 * q, 0, 1)
        block(2 * q + 1, 1, 0)

    if NBLKD % 2 == 1:
        block(NBLKD - 1, 0, 1)

    plsc.subcore_barrier()
    pltpu.sync_copy(
        z_s.at[pl.ds(s * RPT, RPT)],
        out_hbm.at[pl.ds(c * NP + s * RPT, RPT)],
    )


def _prep_body(sdeg_ref, x_ref, dise_ref, rdis_ref, y0_ref):
    deg = jnp.maximum(sdeg_ref[0:NP, 0:1] + sdeg_ref[NP:2 * NP, 0:1] + 1.0,
                      1.0)
    dis = lax.rsqrt(deg)
    dise_ref[...] = dis
    rdis_ref[...] = jnp.sqrt(deg)
    y0_ref[...] = x_ref[...] * dis


_prep_call = pl.pallas_call(
    _prep_body,
    out_shape=(
        jax.ShapeDtypeStruct((NP, 1), jnp.float32),
        jax.ShapeDtypeStruct((NP, 1), jnp.float32),
        jax.ShapeDtypeStruct((NP, D), jnp.float32),
    ),
)


def _scale_body(s_ref, y_ref, dise_ref, ynew_ref):
    dis = dise_ref[...]
    ynew_ref[...] = (s_ref[0:NP, :] + s_ref[NP:2 * NP, :] + y_ref[...]) * (
        dis * dis)


_scale_call = pl.pallas_call(
    _scale_body,
    out_shape=jax.ShapeDtypeStruct((NP, D), jnp.float32),
)


def _final_body(x_ref, t1_ref, t2_ref, t3_ref, t4_ref, rdis_ref, w0t_ref,
                wts_ref, bsum_ref, fw_ref, wct_ref, bc_ref, out_ref):
    rdis = rdis_ref[...]
    acc = jnp.dot(x_ref[...], w0t_ref[...], preferred_element_type=jnp.float32)
    for i, t_ref in enumerate((t1_ref, t2_ref, t3_ref, t4_ref)):
        acc = acc + jnp.dot(t_ref[...] * rdis, wts_ref[i],
                            preferred_element_type=jnp.float32)
    acc = acc + bsum_ref[...]
    h = jnp.maximum(acc, 0.0)
    ew = jnp.exp(fw_ref[...])
    w0 = ew[0, 0] / (ew[0, 0] + ew[0, 1])
    out_ref[...] = jnp.dot(h * w0, wct_ref[...],
                           preferred_element_type=jnp.float32) + bc_ref[...]


_final_call = pl.pallas_call(
    _final_body,
    out_shape=jax.ShapeDtypeStruct((NP, 64), jnp.float32),
)


@jax.jit
def kernel(x, edge_index, W0, b0, W1, b1, W2, b2, W3, b3, W4, b4,
           fusion_weight, Wc, bc):
    pad = EP - E
    rows_p = jnp.concatenate([edge_index[0], jnp.zeros((pad,), jnp.int32)])
    pad_cols = N + (jnp.arange(pad, dtype=jnp.int32) % (NP - N))
    cols_p = jnp.concatenate([edge_index[1], pad_cols])
    rows3 = rows_p.reshape(NBLKS, NB, CH)
    cols3 = cols_p.reshape(NBLKS, NB, CH)
    rc = jnp.stack([rows3, cols3], axis=1)
    x_pad = jnp.pad(x, ((0, NP - N), (0, 0)))

    w0t = W0.T
    wts = jnp.stack([W1.T, W2.T, W3.T, W4.T])
    bsum = (b0 + b1 + b2 + b3 + b4).reshape(1, D)
    fw = fusion_weight.reshape(1, 2)
    wct = Wc.T
    bc2 = bc.reshape(1, 64)

    sdeg = _deg_kernel(rc)
    dise, rdis, y = _prep_call(sdeg, x_pad)

    ys = []
    for _ in range(4):
        s_out = _spmv_kernel(y, rc)
        y = _scale_call(s_out, y, dise)
        ys.append(y)

    logits = _final_call(x_pad, ys[0], ys[1], ys[2], ys[3], rdis,
                        w0t, wts, bsum, fw, wct, bc2)
    return logits[:N]


# speedup vs baseline: 1.0514x; 1.0514x over previous
import functools
import jax
import jax.numpy as jnp
from jax import lax
from jax.experimental import pallas as pl
from jax.experimental.pallas import tpu as pltpu
from jax.experimental.pallas import tpu_sc as plsc

N = 10000
NP = 10112
D = 128
CH = 112
NB = 6
NBLK0 = 21
NBLK1 = 9
NBLKD = 15
NBLKS = 16 * (NBLK0 + NBLK1)
NBLKS_PAD = NBLKS + 16
E = 320000
EP = NBLKS * NB * CH
RPT = NP // 16

_mesh = plsc.VectorSubcoreMesh(core_axis_name="c", subcore_axis_name="s")


@functools.partial(
    pl.kernel,
    out_type=jax.ShapeDtypeStruct((2 * NP, D), jnp.float32),
    mesh=_mesh,
    scratch_types=[
        pltpu.VMEM((2, 2, NB, CH), jnp.int32),
        pltpu.VMEM((CH, D), jnp.float32),
        pltpu.VMEM((CH, D), jnp.float32),
        pltpu.VMEM_SHARED((NP, D), jnp.float32),
        pltpu.SemaphoreType.DMA,
        pltpu.SemaphoreType.DMA,
        pltpu.SemaphoreType.DMA,
    ],
)
def _spmv_kernel(y_hbm, rc_hbm, out_hbm, rcblk, gbufa, gbufb,
                 z_s, sema, semb, semr):
    c = lax.axis_index("c")
    s = lax.axis_index("s")
    nblk = jnp.where(c == 0, NBLK0, NBLK1)
    blkbase = jnp.where(c == 0, s * NBLK0, 16 * NBLK0 + s * NBLK1)

    zero16 = jnp.zeros((16,), jnp.float32)

    @pl.loop(0, CH)
    def _(i):
        for q in range(8):
            gbufa[i, pl.ds(16 * q, 16)] = zero16

    @pl.loop(0, 5)
    def _(k):
        pltpu.sync_copy(gbufa.at[pl.ds(0, 112)],
                        z_s.at[pl.ds(s * RPT + k * 112, 112)])

    pltpu.sync_copy(gbufa.at[pl.ds(0, 72)], z_s.at[pl.ds(s * RPT + 560, 72)])

    plsc.subcore_barrier()

    gbufs = (gbufa, gbufb)
    sems = (sema, semb)

    def wait_gather(buf, sem):
        pltpu.make_async_copy(y_hbm.at[rcblk.at[0, 0, 0]], buf, sem).wait()

    def load_rcblk(b, p, copy):
        copy(rc_hbm.at[0, b], rcblk.at[p, 0])
        copy(rc_hbm.at[1, b], rcblk.at[p, 1])

    def wait_rcblk():
        pltpu.make_async_copy(rc_hbm.at[0, blkbase], rcblk.at[0, 0],
                              semr).wait()
        pltpu.make_async_copy(rc_hbm.at[1, blkbase], rcblk.at[0, 1],
                              semr).wait()

    load_rcblk(blkbase, 0, pltpu.sync_copy)
    load_rcblk(blkbase + 1, 1, lambda a, b: pltpu.async_copy(a, b, semr))
    pltpu.async_copy(y_hbm.at[rcblk.at[0, 0, 0]], gbufa, sema)

    def block(b, cur, nxt):
        for t in range(NB):
            if t < NB - 1:
                pltpu.async_copy(y_hbm.at[rcblk.at[cur, 0, t + 1]],
                                 gbufs[(t + 1) % 2], sems[(t + 1) % 2])
            else:
                @pl.when(b < nblk - 1)
                def _():
                    wait_rcblk()
                    pltpu.async_copy(y_hbm.at[rcblk.at[nxt, 0, 0]],
                                     gbufs[(t + 1) % 2], sems[(t + 1) % 2])
            wait_gather(gbufs[t % 2], sems[t % 2])
            pltpu.sync_copy(gbufs[t % 2], z_s.at[rcblk.at[cur, 1, t]],
                            add=True)

        @pl.when(b + 2 < nblk)
        def _():
            load_rcblk(blkbase + b + 2, cur,
                       lambda a, b2: pltpu.async_copy(a, b2, semr))

    @pl.loop(0, (nblk - 1) // 2)
    def _(q):
        block(2 * q, 0, 1)
        block(2 * q + 1, 1, 0)

    block(nblk - 1, 0, 1)

    plsc.subcore_barrier()
    pltpu.sync_copy(
        z_s.at[pl.ds(s * RPT, RPT)],
        out_hbm.at[pl.ds(c * NP + s * RPT, RPT)],
    )


@functools.partial(
    pl.kernel,
    out_type=jax.ShapeDtypeStruct((2 * NP, D), jnp.float32),
    mesh=_mesh,
    scratch_types=[
        pltpu.VMEM((2, NB, CH), jnp.int32),
        pltpu.VMEM((CH, D), jnp.float32),
        pltpu.VMEM_SHARED((NP, D), jnp.float32),
        pltpu.SemaphoreType.DMA,
    ],
)
def _deg_kernel(rc_hbm, out_hbm, rcblk, gones, z_s, semr):
    c = lax.axis_index("c")
    s = lax.axis_index("s")
    nblk = NBLKD
    blkbase = (c * 16 + s) * NBLKD

    zero16 = jnp.zeros((16,), jnp.float32)
    ones16 = jnp.ones((16,), jnp.float32)

    @pl.loop(0, CH)
    def _(i):
        for q in range(8):
            gones[i, pl.ds(16 * q, 16)] = zero16

    @pl.loop(0, 5)
    def _(k):
        pltpu.sync_copy(gones.at[pl.ds(0, 112)],
                        z_s.at[pl.ds(s * RPT + k * 112, 112)])

    pltpu.sync_copy(gones.at[pl.ds(0, 72)], z_s.at[pl.ds(s * RPT + 560, 72)])

    @pl.loop(0, CH)
    def _(i):
        gones[i, pl.ds(0, 16)] = ones16

    plsc.subcore_barrier()

    def wait_rcblk():
        pltpu.make_async_copy(rc_hbm.at[1, blkbase], rcblk.at[0], semr).wait()

    pltpu.sync_copy(rc_hbm.at[1, blkbase], rcblk.at[0])
    pltpu.async_copy(rc_hbm.at[1, blkbase + 1], rcblk.at[1], semr)

    def block(b, cur, nxt):
        for t in range(NB):
            pltpu.sync_copy(gones, z_s.at[rcblk.at[cur, t]], add=True)

        @pl.when(b < nblk - 2)
        def _():
            wait_rcblk()
            pltpu.async_copy(rc_hbm.at[1, blkbase + b + 2], rcblk.at[cur],
                             semr)

        @pl.when(b == nblk - 2)
        def _():
            wait_rcblk()

    @pl.loop(0, NBLKD // 2)
    def _(q):
        block(2 * q, 0, 1)
        block(2 * q + 1, 1, 0)

    if NBLKD % 2 == 1:
        block(NBLKD - 1, 0, 1)

    plsc.subcore_barrier()
    pltpu.sync_copy(
        z_s.at[pl.ds(s * RPT, RPT)],
        out_hbm.at[pl.ds(c * NP + s * RPT, RPT)],
    )


def _prep_body(sdeg_ref, x_ref, dise_ref, rdis_ref, y0_ref):
    deg = jnp.maximum(sdeg_ref[0:NP, 0:1] + sdeg_ref[NP:2 * NP, 0:1] + 1.0,
                      1.0)
    dis = lax.rsqrt(deg)
    dise_ref[...] = dis
    rdis_ref[...] = jnp.sqrt(deg)
    y0_ref[...] = x_ref[...] * dis


_prep_call = pl.pallas_call(
    _prep_body,
    out_shape=(
        jax.ShapeDtypeStruct((NP, 1), jnp.float32),
        jax.ShapeDtypeStruct((NP, 1), jnp.float32),
        jax.ShapeDtypeStruct((NP, D), jnp.float32),
    ),
)


def _scale_body(s_ref, y_ref, dise_ref, ynew_ref):
    dis = dise_ref[...]
    ynew_ref[...] = (s_ref[0:NP, :] + s_ref[NP:2 * NP, :] + y_ref[...]) * (
        dis * dis)


_scale_call = pl.pallas_call(
    _scale_body,
    out_shape=jax.ShapeDtypeStruct((NP, D), jnp.float32),
)


def _final_body(x_ref, t1_ref, t2_ref, t3_ref, t4_ref, rdis_ref, w0t_ref,
                wts_ref, bsum_ref, fw_ref, wct_ref, bc_ref, out_ref):
    rdis = rdis_ref[...]
    acc = jnp.dot(x_ref[...], w0t_ref[...], preferred_element_type=jnp.float32)
    for i, t_ref in enumerate((t1_ref, t2_ref, t3_ref, t4_ref)):
        acc = acc + jnp.dot(t_ref[...] * rdis, wts_ref[i],
                            preferred_element_type=jnp.float32)
    acc = acc + bsum_ref[...]
    h = jnp.maximum(acc, 0.0)
    ew = jnp.exp(fw_ref[...])
    w0 = ew[0, 0] / (ew[0, 0] + ew[0, 1])
    out_ref[...] = jnp.dot(h * w0, wct_ref[...],
                           preferred_element_type=jnp.float32) + bc_ref[...]


_final_call = pl.pallas_call(
    _final_body,
    out_shape=jax.ShapeDtypeStruct((NP, 64), jnp.float32),
)


@jax.jit
def kernel(x, edge_index, W0, b0, W1, b1, W2, b2, W3, b3, W4, b4,
           fusion_weight, Wc, bc):
    pad = EP - E
    pad_blk = jnp.stack([jnp.zeros((pad,), jnp.int32),
                         N + (jnp.arange(pad, dtype=jnp.int32) % (NP - N))])
    rc = jnp.concatenate([edge_index, pad_blk], axis=1).reshape(
        2, NBLKS, NB, CH)
    x_pad = jnp.pad(x, ((0, NP - N), (0, 0)))

    w0t = W0.T
    wts = jnp.stack([W1.T, W2.T, W3.T, W4.T])
    bsum = (b0 + b1 + b2 + b3 + b4).reshape(1, D)
    fw = fusion_weight.reshape(1, 2)
    wct = Wc.T
    bc2 = bc.reshape(1, 64)

    sdeg = _deg_kernel(rc)
    dise, rdis, y = _prep_call(sdeg, x_pad)

    ys = []
    for _ in range(4):
        s_out = _spmv_kernel(y, rc)
        y = _scale_call(s_out, y, dise)
        ys.append(y)

    logits = _final_call(x_pad, ys[0], ys[1], ys[2], ys[3], rdis,
                        w0t, wts, bsum, fw, wct, bc2)
    return logits[:N]
